# baseline (speedup 1.0000x reference)
import jax
import jax.numpy as jnp
from jax.experimental import pallas as pl

_INPUT_LEN = 200
_EMBED_DIM = 64
_BATCH = 4096
_FLAT = _INPUT_LEN * _EMBED_DIM
_BB = 256


def _bcast_body(pos_ref, out_ref):
    out_ref[...] = jnp.broadcast_to(pos_ref[...], out_ref.shape)


def kernel(x, pos_table):
    del x
    pos_flat = pos_table.reshape(1, _FLAT)
    out = pl.pallas_call(
        _bcast_body,
        grid=(_BATCH // _BB,),
        in_specs=[pl.BlockSpec((1, _FLAT), lambda i: (0, 0))],
        out_specs=pl.BlockSpec((_BB, _FLAT), lambda i: (i, 0)),
        out_shape=jax.ShapeDtypeStruct((_BATCH, _FLAT), jnp.float32),
    )(pos_flat)
    return out.reshape(_BATCH, _INPUT_LEN, _EMBED_DIM)


# speedup vs baseline: 1.0024x; 1.0024x over previous
import jax
import jax.numpy as jnp
from jax.experimental import pallas as pl
from jax.experimental.pallas import tpu as pltpu

_INPUT_LEN = 200
_EMBED_DIM = 64
_BATCH = 4096
_FLAT = _INPUT_LEN * _EMBED_DIM
_TR = 256
_NB = _BATCH // _TR


def _bcast_body(pos_ref, out_ref, tile_ref, sem):
    tile_ref[...] = jnp.broadcast_to(pos_ref[...], tile_ref.shape)
    copies = [
        pltpu.make_async_copy(tile_ref, out_ref.at[pl.ds(j * _TR, _TR), :], sem)
        for j in range(_NB)
    ]
    for c in copies:
        c.start()
    for c in copies:
        c.wait()


def kernel(x, pos_table):
    del x
    pos_flat = pos_table.reshape(1, _FLAT)
    out = pl.pallas_call(
        _bcast_body,
        in_specs=[pl.BlockSpec((1, _FLAT), lambda: (0, 0))],
        out_specs=pl.BlockSpec(memory_space=pl.ANY),
        out_shape=jax.ShapeDtypeStruct((_BATCH, _FLAT), jnp.float32),
        scratch_shapes=[
            pltpu.VMEM((_TR, _FLAT), jnp.float32),
            pltpu.SemaphoreType.DMA,
        ],
    )(pos_flat)
    return out.reshape(_BATCH, _INPUT_LEN, _EMBED_DIM)
